# baseline (speedup 1.0000x reference)
import functools

import jax
import jax.numpy as jnp
from jax import lax
from jax.experimental import pallas as pl
from jax.experimental.pallas import tpu as pltpu
from jax.experimental.pallas import tpu_sc as plsc

N = 10000
D = 128
E = 320000
NC = 2
NS = 16
NW = NC * NS
EPT = E // NW
C = 125
NCHUNK = EPT // C
NP = 10240
RPT = NP // NS
DEGP = 10240
NBUF = 8
DPT = DEGP // NS

_mesh = plsc.VectorSubcoreMesh(
    core_axis_name="c", subcore_axis_name="s", num_cores=NC, num_subcores=NS)


def _deg_body(dst_hbm, zeros_hbm, out_hbm, idx_v, ones_v, deg_sh,
              d0, d1, d2, d3, d4):
    dsem = (d0, d1, d2, d3, d4)
    c = lax.axis_index("c")
    s = lax.axis_index("s")
    w = c * NS + s
    pltpu.sync_copy(dst_hbm.at[w], idx_v)
    for k in range(8):
        ones_v[pl.ds(k * 16, 16)] = jnp.ones((16,), jnp.float32)
    pltpu.sync_copy(zeros_hbm.at[pl.ds(s * DPT, DPT)],
                    deg_sh.at[pl.ds(s * DPT, DPT)])
    plsc.subcore_barrier()

    def body(g, carry):
        base = g * 5
        ds = [pltpu.async_copy(ones_v.at[pl.ds(0, C)],
                               deg_sh.at[idx_v.at[base + b]],
                               dsem[b], add=True) for b in range(5)]
        for d in ds:
            d.wait()
        return carry

    lax.fori_loop(0, NCHUNK // 5, body, 0)
    plsc.subcore_barrier()
    pltpu.sync_copy(deg_sh.at[pl.ds(s * DPT, DPT)],
                    out_hbm.at[c, pl.ds(s * DPT, DPT)])


_deg_call = pl.kernel(
    _deg_body,
    out_type=jax.ShapeDtypeStruct((NC, DEGP), jnp.float32),
    mesh=_mesh,
    compiler_params=pltpu.CompilerParams(use_tc_tiling_on_sc=False),
    scratch_types=[
        pltpu.VMEM((NCHUNK, C), jnp.int32),
        pltpu.VMEM((128,), jnp.float32),
        pltpu.VMEM_SHARED((DEGP,), jnp.float32),
    ] + [pltpu.SemaphoreType.DMA] * 5,
)


DH = D // 2


def _agg_body(glo_hbm, ghi_hbm, src_hbm, dst_hbm, zeros_hbm,
              outlo_hbm, outhi_hbm,
              srcb, dstb,
              r0, r1, r2, r3, r4, r5, r6, r7,
              g0, g1, g2, g3, g4, g5, g6, g7,
              s0, s1, s2, s3, s4, s5, s6, s7,
              acc_sh):
    rows = (r0, r1, r2, r3, r4, r5, r6, r7)
    gsem = (g0, g1, g2, g3, g4, g5, g6, g7)
    ssem = (s0, s1, s2, s3, s4, s5, s6, s7)
    c = lax.axis_index("c")
    s = lax.axis_index("s")
    w = c * NS + s
    pltpu.sync_copy(src_hbm.at[w], srcb)
    pltpu.sync_copy(dst_hbm.at[w], dstb)

    def half(g_hbm, out_hbm):
        pltpu.sync_copy(zeros_hbm.at[pl.ds(s * RPT, RPT)],
                        acc_sh.at[pl.ds(s * RPT, RPT)])
        plsc.subcore_barrier()

        for b in range(NBUF):
            pltpu.async_copy(g_hbm.at[srcb.at[b]], rows[b], gsem[b])

        def group(g, carry):
            for b in range(NBUF):
                j = g * NBUF + b
                pltpu.make_async_copy(g_hbm.at[srcb.at[j]], rows[b],
                                      gsem[b]).wait()
                pltpu.async_copy(rows[b], acc_sh.at[dstb.at[j]], ssem[b],
                                 add=True)
                pltpu.make_async_copy(rows[b], acc_sh.at[dstb.at[j]],
                                      ssem[b]).wait()
                pltpu.async_copy(g_hbm.at[srcb.at[j + NBUF]], rows[b],
                                 gsem[b])
            return carry

        lax.fori_loop(0, NCHUNK // NBUF - 1, group, 0)
        for b in range(NBUF):
            j = NCHUNK - NBUF + b
            pltpu.make_async_copy(g_hbm.at[srcb.at[j]], rows[b],
                                  gsem[b]).wait()
            pltpu.async_copy(rows[b], acc_sh.at[dstb.at[j]], ssem[b],
                             add=True)
            pltpu.make_async_copy(rows[b], acc_sh.at[dstb.at[j]],
                                  ssem[b]).wait()
        plsc.subcore_barrier()
        pltpu.sync_copy(acc_sh.at[pl.ds(s * RPT, RPT)],
                        out_hbm.at[c, pl.ds(s * RPT, RPT)])

    half(glo_hbm, outlo_hbm)
    half(ghi_hbm, outhi_hbm)


_agg_call = pl.kernel(
    _agg_body,
    out_type=(jax.ShapeDtypeStruct((NC, NP, DH), jnp.float32),
              jax.ShapeDtypeStruct((NC, NP, DH), jnp.float32)),
    mesh=_mesh,
    compiler_params=pltpu.CompilerParams(use_tc_tiling_on_sc=False),
    scratch_types=(
        [pltpu.VMEM((NCHUNK, C), jnp.int32)] * 2
        + [pltpu.VMEM((C, DH), jnp.float32)] * NBUF
        + [pltpu.SemaphoreType.DMA] * (2 * NBUF)
        + [pltpu.VMEM_SHARED((NP, DH), jnp.float32)]
    ),
)


def _tc1_body(x_ref, w_ref, dinv_ref, g_ref):
    h = jnp.dot(x_ref[...], w_ref[...], preferred_element_type=jnp.float32)
    g_ref[...] = h * dinv_ref[...]


def _tc2_body(aplo_ref, aphi_ref, g1_ref, dinv_ref, b_ref, gam_ref, bet_ref,
              w2_ref, h1_ref, g2_ref):
    dinv = dinv_ref[...]
    a = jnp.concatenate([aplo_ref[0, :N] + aplo_ref[1, :N],
                         aphi_ref[0, :N] + aphi_ref[1, :N]], axis=-1)
    pre = (a + g1_ref[...]) * dinv + b_ref[...]
    m = jnp.mean(pre, axis=0, keepdims=True)
    d = pre - m
    v = jnp.mean(d * d, axis=0, keepdims=True)
    y = d * lax.rsqrt(v + 1e-5) * gam_ref[...] + bet_ref[...]
    h1 = jnp.where(y > 0, y, jnp.exp(jnp.minimum(y, 0.0)) - 1.0)
    h1_ref[...] = h1
    g2_ref[...] = jnp.dot(h1, w2_ref[...],
                          preferred_element_type=jnp.float32) * dinv


def _tc3_body(aplo_ref, aphi_ref, g2_ref, dinv_ref, b_ref, gam_ref, bet_ref,
              h2_ref):
    a = jnp.concatenate([aplo_ref[0, :N] + aplo_ref[1, :N],
                         aphi_ref[0, :N] + aphi_ref[1, :N]], axis=-1)
    pre = (a + g2_ref[...]) * dinv_ref[...] + b_ref[...]
    m = jnp.mean(pre, axis=0, keepdims=True)
    d = pre - m
    v = jnp.mean(d * d, axis=0, keepdims=True)
    y = d * lax.rsqrt(v + 1e-5) * gam_ref[...] + bet_ref[...]
    h2_ref[...] = jnp.where(y > 0, y, jnp.exp(jnp.minimum(y, 0.0)) - 1.0)


_tc1_call = pl.pallas_call(
    _tc1_body,
    out_shape=jax.ShapeDtypeStruct((N, D), jnp.float32),
)

_tc2_call = pl.pallas_call(
    _tc2_body,
    out_shape=(jax.ShapeDtypeStruct((N, D), jnp.float32),
               jax.ShapeDtypeStruct((N, D), jnp.float32)),
)

_tc3_call = pl.pallas_call(
    _tc3_body,
    out_shape=jax.ShapeDtypeStruct((N, D), jnp.float32),
)


@jax.jit
def kernel(x, edge_index, W1, b1, gamma1, beta1, W2, b2, gamma2, beta2):
    src = edge_index[0].reshape(NW, NCHUNK, C)
    dst = edge_index[1].reshape(NW, NCHUNK, C)
    zeros_deg = jnp.zeros((DEGP,), jnp.float32)
    zeros_acc = jnp.zeros((NP, DH), jnp.float32)
    degp = _deg_call(dst, zeros_deg)
    dinv = lax.rsqrt(degp[0, :N] + degp[1, :N] + 1.0)[:, None]

    g1 = _tc1_call(x, W1, dinv)
    ap1lo, ap1hi = _agg_call(g1[:, :DH], g1[:, DH:], src, dst, zeros_acc)
    h1, g2 = _tc2_call(ap1lo, ap1hi, g1, dinv, b1.reshape(1, D),
                       gamma1.reshape(1, D), beta1.reshape(1, D), W2)
    ap2lo, ap2hi = _agg_call(g2[:, :DH], g2[:, DH:], src, dst, zeros_acc)
    h2 = _tc3_call(ap2lo, ap2hi, g2, dinv, b2.reshape(1, D),
                   gamma2.reshape(1, D), beta2.reshape(1, D))
    return (x, h1, h2)


# speedup vs baseline: 1.1901x; 1.1901x over previous
import functools

import jax
import jax.numpy as jnp
from jax import lax
from jax.experimental import pallas as pl
from jax.experimental.pallas import tpu as pltpu
from jax.experimental.pallas import tpu_sc as plsc

N = 10000
D = 128
DH = D // 2
E = 320000
NC = 2
NS = 16
NW = NC * NS
EPT = E // NW
C = 80
NCHUNK = EPT // C
NBUF = 5
NP = 10240
RPT = NP // NS
DEGP = 10240
DPT = DEGP // NS

_mesh = plsc.VectorSubcoreMesh(
    core_axis_name="c", subcore_axis_name="s", num_cores=NC, num_subcores=NS)

_sc_params = pltpu.CompilerParams(use_tc_tiling_on_sc=False)


def _deg_body(edges_hbm, zeros_hbm, out_hbm, idx_v, ones_v, deg_sh,
              d0, d1, d2, d3, d4):
    dsem = (d0, d1, d2, d3, d4)
    c = lax.axis_index("c")
    s = lax.axis_index("s")
    w = c * NS + s
    pltpu.sync_copy(edges_hbm.at[1, w], idx_v)
    for k in range(C // 16):
        ones_v[pl.ds(k * 16, 16)] = jnp.ones((16,), jnp.float32)
    pltpu.sync_copy(zeros_hbm.at[pl.ds(s * DPT, DPT)],
                    deg_sh.at[pl.ds(s * DPT, DPT)])
    plsc.subcore_barrier()

    def body(g, carry):
        base = g * NBUF
        ds = [pltpu.async_copy(ones_v, deg_sh.at[idx_v.at[base + b]],
                               dsem[b], add=True) for b in range(NBUF)]
        for d in ds:
            d.wait()
        return carry

    lax.fori_loop(0, NCHUNK // NBUF, body, 0)
    plsc.subcore_barrier()
    pltpu.sync_copy(deg_sh.at[pl.ds(s * DPT, DPT)],
                    out_hbm.at[c, pl.ds(s * DPT, DPT)])


_deg_call = pl.kernel(
    _deg_body,
    out_type=jax.ShapeDtypeStruct((NC, DEGP), jnp.float32),
    mesh=_mesh,
    compiler_params=_sc_params,
    scratch_types=[
        pltpu.VMEM((NCHUNK, C), jnp.int32),
        pltpu.VMEM((C,), jnp.float32),
        pltpu.VMEM_SHARED((DEGP,), jnp.float32),
    ] + [pltpu.SemaphoreType.DMA] * NBUF,
)


def _agg_body(glo_hbm, ghi_hbm, edges_hbm, zeros_hbm, out_hbm,
              srcb, dstb, r0, r1, r2, r3, r4,
              g0, g1, g2, g3, g4, s0, s1, s2, s3, s4,
              acc_sh):
    rows = (r0, r1, r2, r3, r4)
    gsem = (g0, g1, g2, g3, g4)
    ssem = (s0, s1, s2, s3, s4)
    c = lax.axis_index("c")
    s = lax.axis_index("s")
    w = c * NS + s
    pltpu.sync_copy(edges_hbm.at[0, w], srcb)
    pltpu.sync_copy(edges_hbm.at[1, w], dstb)

    def half(g_hbm, h):
        pltpu.sync_copy(zeros_hbm.at[pl.ds(s * RPT, RPT)],
                        acc_sh.at[pl.ds(s * RPT, RPT)])
        plsc.subcore_barrier()

        for b in range(NBUF):
            pltpu.async_copy(g_hbm.at[srcb.at[b]], rows[b], gsem[b])

        def group(g, carry):
            for b in range(NBUF):
                j = g * NBUF + b
                pltpu.make_async_copy(g_hbm.at[srcb.at[j]], rows[b],
                                      gsem[b]).wait()
                pltpu.async_copy(rows[b], acc_sh.at[dstb.at[j]], ssem[b],
                                 add=True)
                pltpu.make_async_copy(rows[b], acc_sh.at[dstb.at[j]],
                                      ssem[b]).wait()
                pltpu.async_copy(g_hbm.at[srcb.at[j + NBUF]], rows[b],
                                 gsem[b])
            return carry

        lax.fori_loop(0, NCHUNK // NBUF - 1, group, 0)
        for b in range(NBUF):
            j = NCHUNK - NBUF + b
            pltpu.make_async_copy(g_hbm.at[srcb.at[j]], rows[b],
                                  gsem[b]).wait()
            pltpu.async_copy(rows[b], acc_sh.at[dstb.at[j]], ssem[b],
                             add=True)
            pltpu.make_async_copy(rows[b], acc_sh.at[dstb.at[j]],
                                  ssem[b]).wait()
        plsc.subcore_barrier()
        pltpu.sync_copy(acc_sh.at[pl.ds(s * RPT, RPT)],
                        out_hbm.at[c, pl.ds(s * RPT, RPT),
                                   pl.ds(h * DH, DH)])

    half(glo_hbm, 0)
    half(ghi_hbm, 1)


_agg_call = pl.kernel(
    _agg_body,
    out_type=jax.ShapeDtypeStruct((NC, NP, D), jnp.float32),
    mesh=_mesh,
    compiler_params=_sc_params,
    scratch_types=(
        [pltpu.VMEM((NCHUNK, C), jnp.int32)] * 2
        + [pltpu.VMEM((C, DH), jnp.float32)] * NBUF
        + [pltpu.SemaphoreType.DMA] * (2 * NBUF)
        + [pltpu.VMEM_SHARED((NP, DH), jnp.float32)]
    ),
)


def _tc1_body(x_ref, w_ref, dinv_ref, g_ref):
    h = jnp.dot(x_ref[...], w_ref[...], preferred_element_type=jnp.float32)
    g_ref[...] = h * dinv_ref[...]


def _tc2_body(ap_ref, g1_ref, dinv_ref, b_ref, gam_ref, bet_ref, w2_ref,
              h1_ref, g2_ref):
    dinv = dinv_ref[...]
    a = ap_ref[0, :N] + ap_ref[1, :N]
    pre = (a + g1_ref[...]) * dinv + b_ref[...]
    m = jnp.mean(pre, axis=0, keepdims=True)
    d = pre - m
    v = jnp.mean(d * d, axis=0, keepdims=True)
    y = d * lax.rsqrt(v + 1e-5) * gam_ref[...] + bet_ref[...]
    h1 = jnp.where(y > 0, y, jnp.exp(jnp.minimum(y, 0.0)) - 1.0)
    h1_ref[...] = h1
    g2_ref[...] = jnp.dot(h1, w2_ref[...],
                          preferred_element_type=jnp.float32) * dinv


def _tc3_body(ap_ref, g2_ref, dinv_ref, b_ref, gam_ref, bet_ref, h2_ref):
    a = ap_ref[0, :N] + ap_ref[1, :N]
    pre = (a + g2_ref[...]) * dinv_ref[...] + b_ref[...]
    m = jnp.mean(pre, axis=0, keepdims=True)
    d = pre - m
    v = jnp.mean(d * d, axis=0, keepdims=True)
    y = d * lax.rsqrt(v + 1e-5) * gam_ref[...] + bet_ref[...]
    h2_ref[...] = jnp.where(y > 0, y, jnp.exp(jnp.minimum(y, 0.0)) - 1.0)


_tc1_call = pl.pallas_call(
    _tc1_body,
    out_shape=jax.ShapeDtypeStruct((N, D), jnp.float32),
)

_tc2_call = pl.pallas_call(
    _tc2_body,
    out_shape=(jax.ShapeDtypeStruct((N, D), jnp.float32),
               jax.ShapeDtypeStruct((N, D), jnp.float32)),
)

_tc3_call = pl.pallas_call(
    _tc3_body,
    out_shape=jax.ShapeDtypeStruct((N, D), jnp.float32),
)


@jax.jit
def kernel(x, edge_index, W1, b1, gamma1, beta1, W2, b2, gamma2, beta2):
    es = edge_index.reshape(2, NW, NCHUNK, C)
    zeros_deg = jnp.zeros((DEGP,), jnp.float32)
    zeros_acc = jnp.zeros((NP, DH), jnp.float32)

    degp = _deg_call(es, zeros_deg)
    dinv = lax.rsqrt(degp[0, :N] + degp[1, :N] + 1.0)[:, None]

    g1 = _tc1_call(x, W1, dinv)
    ap1 = _agg_call(g1[:, :DH], g1[:, DH:], es, zeros_acc)
    h1, g2 = _tc2_call(ap1, g1, dinv, b1.reshape(1, D),
                       gamma1.reshape(1, D), beta1.reshape(1, D), W2)
    ap2 = _agg_call(g2[:, :DH], g2[:, DH:], es, zeros_acc)
    h2 = _tc3_call(ap2, g2, dinv, b2.reshape(1, D),
                   gamma2.reshape(1, D), beta2.reshape(1, D))
    return (x, h1, h2)
